# baseline (speedup 1.0000x reference)
import dataclasses
import functools

import jax
import jax.numpy as jnp
from jax import lax
from jax.experimental import pallas as pl
from jax.experimental.pallas import tpu as pltpu
from jax.experimental.pallas import tpu_sc as plsc

NC, NS, L = 2, 16, 16


def _sc_compiler_params():
    cp = pltpu.CompilerParams()
    if "needs_layout_passes" in pltpu.CompilerParams.__dataclass_fields__:
        cp = dataclasses.replace(cp, needs_layout_passes=False)
    return cp


def _make_deg_kernel(E, N):
    NW = NC * NS
    EPW = E // NW
    NV = EPW // L
    REM = EPW - NV * L
    mesh = plsc.VectorSubcoreMesh(core_axis_name="c", subcore_axis_name="s")

    @functools.partial(
        pl.kernel,
        out_type=jax.ShapeDtypeStruct((NW, N), jnp.float32),
        mesh=mesh,
        compiler_params=_sc_compiler_params(),
        scratch_types=[
            pltpu.VMEM((EPW + L,), jnp.int32),
            pltpu.VMEM((N,), jnp.float32),
        ],
    )
    def deg_kernel(dst_hbm, out_hbm, idx_v, hist_v):
        wid = lax.axis_index("s") * NC + lax.axis_index("c")
        base = wid * EPW
        pltpu.sync_copy(dst_hbm.at[pl.ds(base, EPW)], idx_v.at[pl.ds(0, EPW)])
        zf = jnp.zeros((L,), jnp.float32)
        idx_v[pl.ds(EPW, L)] = jnp.zeros((L,), jnp.int32)

        @pl.loop(0, N, step=L)
        def _(i):
            hist_v[pl.ds(i, L)] = zf

        ones = jnp.ones((L,), jnp.float32)

        @pl.loop(0, NV * L, step=L)
        def _(i):
            plsc.addupdate_scatter(hist_v, [idx_v[pl.ds(i, L)]], ones)

        if REM:
            mask = lax.iota(jnp.int32, L) < REM
            plsc.addupdate_scatter(hist_v, [idx_v[pl.ds(NV * L, L)]], ones,
                                   mask=mask)
        pltpu.sync_copy(hist_v, out_hbm.at[wid])

    return deg_kernel


def _make_agg_kernel(EP, N, F):
    C = 128
    NR = EP // C
    assert NR % NS == 0
    RPT = NR // NS
    assert RPT % 16 == 0
    HR = RPT // 2
    HPAIR = HR // 2
    NACC = N + 8
    RPW = ((N + NS - 1) // NS + 7) // 8 * 8
    RPW_LAST = N - RPW * (NS - 1)
    assert RPW_LAST > 0 and RPW_LAST % 8 == 0
    mesh = plsc.VectorSubcoreMesh(core_axis_name="c", subcore_axis_name="s")

    @functools.partial(
        pl.kernel,
        out_type=[
            jax.ShapeDtypeStruct((N, F), jnp.float32),
            jax.ShapeDtypeStruct((N, F), jnp.float32),
        ],
        mesh=mesh,
        compiler_params=_sc_compiler_params(),
        scratch_types=[
            pltpu.VMEM((HR, C), jnp.int32),
            pltpu.VMEM((HR, C), jnp.int32),
            pltpu.VMEM((C, F), jnp.float32),
            pltpu.VMEM((C, F), jnp.float32),
            pltpu.VMEM_SHARED((NACC, F), jnp.float32),
            pltpu.SemaphoreType.DMA,
            pltpu.SemaphoreType.DMA,
        ],
    )
    def agg_kernel(xs_a, xs_b, src_hbm, dst_hbm, zero_hbm, agg_a, agg_b,
                   sall, dall, rows0, rows1, acc, sem0, sem1):
        cid = lax.axis_index("c")
        sid = lax.axis_index("s")
        roff = pl.multiple_of(sid * RPW, 8)

        @pl.when(sid < NS - 1)
        def _():
            pltpu.sync_copy(zero_hbm.at[pl.ds(roff, RPW)],
                            acc.at[pl.ds(roff, RPW)])

        @pl.when(sid == NS - 1)
        def _():
            loff = pl.multiple_of((NS - 1) * RPW, 8)
            pltpu.sync_copy(zero_hbm.at[pl.ds(loff, RPW_LAST)],
                            acc.at[pl.ds(loff, RPW_LAST)])

        plsc.subcore_barrier()

        def run(table, out):
            for h in range(RPT // HR):
                hoff = pl.multiple_of(sid * RPT + h * HR, 8)
                pltpu.sync_copy(src_hbm.at[pl.ds(hoff, HR)], sall)
                pltpu.sync_copy(dst_hbm.at[pl.ds(hoff, HR)], dall)

                @pl.loop(0, HR)
                def _(j):
                    pltpu.async_copy(table.at[sall.at[j]], rows0, sem0).wait()
                    pltpu.sync_copy(rows0, acc.at[dall.at[j]], add=True)

            plsc.subcore_barrier()

            @pl.when(sid < NS - 1)
            def _():
                pltpu.sync_copy(acc.at[pl.ds(roff, RPW)],
                                out.at[pl.ds(roff, RPW)])

            @pl.when(sid == NS - 1)
            def _():
                loff = pl.multiple_of((NS - 1) * RPW, 8)
                pltpu.sync_copy(acc.at[pl.ds(loff, RPW_LAST)],
                                out.at[pl.ds(loff, RPW_LAST)])

        @pl.when(cid == 0)
        def _():
            run(xs_a, agg_a)

        @pl.when(cid == 1)
        def _():
            run(xs_b, agg_b)

    return agg_kernel


def _dinv_call(degp, N):
    def body(degp_ref, dinv_ref):
        deg = jnp.sum(degp_ref[...], axis=0, keepdims=True) + 1.0
        dinv_ref[...] = lax.rsqrt(deg)

    return pl.pallas_call(
        body, out_shape=jax.ShapeDtypeStruct((1, N), jnp.float32))(degp)


def _prologue_call(x, dinv_col, N, F):
    def body(x_ref, dv_ref, a_ref, b_ref):
        xs = x_ref[...] * dv_ref[...]
        a_ref[...] = xs[:, :F]
        b_ref[...] = xs[:, F:]

    return pl.pallas_call(
        body,
        out_shape=[jax.ShapeDtypeStruct((N, F), jnp.float32),
                   jax.ShapeDtypeStruct((N, F), jnp.float32)])(x, dinv_col)


def _epilogue_call(x, agg_a, agg_b, dinv_col, batch3, W1, b1, W2, b2,
                   N, R, G, HID):
    nblk = N // R

    def body(x_ref, aa_ref, ab_ref, dv_ref, b_ref, W1_ref, b1_ref, W2_ref,
             b2_ref, node_ref, graph_ref, sums_ref, cnts_ref):
        i = pl.program_id(0)
        dv = dv_ref[...]
        agg = jnp.concatenate([aa_ref[...], ab_ref[...]], axis=1)
        Z = dv * (agg + dv * x_ref[...])
        H = jnp.tanh(
            jnp.dot(Z, W1_ref[...], preferred_element_type=jnp.float32)
            + b1_ref[...])
        node_ref[...] = H
        bat = b_ref[0]
        gid = lax.broadcasted_iota(jnp.int32, (G, R), 0)
        onehot = (bat == gid).astype(jnp.float32)
        psum = jnp.dot(onehot, H, preferred_element_type=jnp.float32)
        pcnt = jnp.sum(onehot, axis=1, keepdims=True)

        @pl.when(i == 0)
        def _():
            sums_ref[...] = psum
            cnts_ref[...] = jnp.broadcast_to(pcnt, (G, 128))

        @pl.when(i > 0)
        def _():
            sums_ref[...] += psum
            cnts_ref[...] += jnp.broadcast_to(pcnt, (G, 128))

        @pl.when(i == nblk - 1)
        def _():
            cnt = jnp.maximum(cnts_ref[:, :1], 1.0)
            mean = sums_ref[...] / cnt
            graph_ref[...] = jnp.tanh(
                jnp.dot(mean, W2_ref[...], preferred_element_type=jnp.float32)
                + b2_ref[...])

    F = agg_a.shape[1]
    IN = x.shape[1]
    return pl.pallas_call(
        body,
        grid=(nblk,),
        in_specs=[
            pl.BlockSpec((R, IN), lambda i: (i, 0)),
            pl.BlockSpec((R, F), lambda i: (i, 0)),
            pl.BlockSpec((R, F), lambda i: (i, 0)),
            pl.BlockSpec((R, 1), lambda i: (i, 0)),
            pl.BlockSpec((1, 1, R), lambda i: (i, 0, 0)),
            pl.BlockSpec((IN, HID), lambda i: (0, 0)),
            pl.BlockSpec((1, HID), lambda i: (0, 0)),
            pl.BlockSpec((HID, HID), lambda i: (0, 0)),
            pl.BlockSpec((1, HID), lambda i: (0, 0)),
        ],
        out_specs=[
            pl.BlockSpec((R, HID), lambda i: (i, 0)),
            pl.BlockSpec((G, HID), lambda i: (0, 0)),
        ],
        out_shape=[jax.ShapeDtypeStruct((N, HID), jnp.float32),
                   jax.ShapeDtypeStruct((G, HID), jnp.float32)],
        scratch_shapes=[pltpu.VMEM((G, HID), jnp.float32),
                        pltpu.VMEM((G, 128), jnp.float32)],
    )(x, agg_a, agg_b, dinv_col, batch3, W1, b1, W2, b2)


def _impl(x, edge_index, batch, W1, b1, W2, b2):
    N, IN = x.shape
    E = edge_index.shape[1]
    HID = W1.shape[1]
    G = 64
    F = IN // 2
    R = 1000

    ei = edge_index.astype(jnp.int32)
    src, dst = ei[0], ei[1]

    degp = _make_deg_kernel(E, N)(dst)
    dinv_col = _dinv_call(degp, N).reshape(N, 1)
    xs_a, xs_b = _prologue_call(x, dinv_col, N, F)
    zeros = jnp.zeros((N, F), jnp.float32)
    EP = ((E + 128 * NS * 8 - 1) // (128 * NS * 8)) * 128 * NS * 8
    src2 = jnp.concatenate(
        [src, jnp.zeros((EP - E,), jnp.int32)]).reshape(EP // 128, 128)
    dst2 = jnp.concatenate(
        [dst, jnp.full((EP - E,), N, jnp.int32)]).reshape(EP // 128, 128)
    agg_a, agg_b = _make_agg_kernel(EP, N, F)(xs_a, xs_b, src2, dst2, zeros)

    batch3 = batch.astype(jnp.int32).reshape(N // R, 1, R)
    node, graph = _epilogue_call(
        x, agg_a, agg_b, dinv_col, batch3,
        W1, b1.reshape(1, HID), W2, b2.reshape(1, HID), N, R, G, HID)
    return (graph, node)


kernel = jax.jit(_impl)


# speedup vs baseline: 1.2583x; 1.2583x over previous
import dataclasses
import functools

import jax
import jax.numpy as jnp
from jax import lax
from jax.experimental import pallas as pl
from jax.experimental.pallas import tpu as pltpu
from jax.experimental.pallas import tpu_sc as plsc

NC, NS, L = 2, 16, 16


def _sc_compiler_params():
    cp = pltpu.CompilerParams()
    if "needs_layout_passes" in pltpu.CompilerParams.__dataclass_fields__:
        cp = dataclasses.replace(cp, needs_layout_passes=False)
    return cp


def _make_deg_kernel(E, N):
    NW = NC * NS
    EPW = E // NW
    NV = EPW // L
    REM = EPW - NV * L
    mesh = plsc.VectorSubcoreMesh(core_axis_name="c", subcore_axis_name="s")

    @functools.partial(
        pl.kernel,
        out_type=jax.ShapeDtypeStruct((NW, N), jnp.float32),
        mesh=mesh,
        compiler_params=_sc_compiler_params(),
        scratch_types=[
            pltpu.VMEM((EPW + L,), jnp.int32),
            pltpu.VMEM((N,), jnp.float32),
        ],
    )
    def deg_kernel(dst_hbm, out_hbm, idx_v, hist_v):
        wid = lax.axis_index("s") * NC + lax.axis_index("c")
        base = wid * EPW
        pltpu.sync_copy(dst_hbm.at[pl.ds(base, EPW)], idx_v.at[pl.ds(0, EPW)])
        zf = jnp.zeros((L,), jnp.float32)
        idx_v[pl.ds(EPW, L)] = jnp.zeros((L,), jnp.int32)

        @pl.loop(0, N, step=L)
        def _(i):
            hist_v[pl.ds(i, L)] = zf

        ones = jnp.ones((L,), jnp.float32)

        @pl.loop(0, NV * L, step=L)
        def _(i):
            plsc.addupdate_scatter(hist_v, [idx_v[pl.ds(i, L)]], ones)

        if REM:
            mask = lax.iota(jnp.int32, L) < REM
            plsc.addupdate_scatter(hist_v, [idx_v[pl.ds(NV * L, L)]], ones,
                                   mask=mask)
        pltpu.sync_copy(hist_v, out_hbm.at[wid])

    return deg_kernel


def _make_agg_kernel(EP, N, F):
    C = 128
    EPS = EP // NS
    NCH = EPS // C
    assert NCH % 4 == 0 and EPS % 8 == 0
    NACC = N + 8
    RPW = ((N + NS - 1) // NS + 7) // 8 * 8
    RPW_LAST = N - RPW * (NS - 1)
    assert RPW_LAST > 0 and RPW_LAST % 8 == 0
    mesh = plsc.VectorSubcoreMesh(core_axis_name="c", subcore_axis_name="s")

    @functools.partial(
        pl.kernel,
        out_type=[
            jax.ShapeDtypeStruct((N, F), jnp.float32),
            jax.ShapeDtypeStruct((N, F), jnp.float32),
        ],
        mesh=mesh,
        compiler_params=_sc_compiler_params(),
        scratch_types=(
            [pltpu.VMEM((C,), jnp.int32)] * 8
            + [pltpu.VMEM((C, F), jnp.float32)] * 2
            + [pltpu.VMEM_SHARED((NACC, F), jnp.float32)]
            + [pltpu.SemaphoreType.DMA] * 6
        ),
    )
    def agg_kernel(xs_a, xs_b, src_hbm, dst_hbm, zero_hbm, agg_a, agg_b,
                   si0, si1, si2, si3, di0, di1, di2, di3, rows0, rows1,
                   acc, semi0, semi1, semi2, semi3, semg0, semg1):
        cid = lax.axis_index("c")
        sid = lax.axis_index("s")
        roff = pl.multiple_of(sid * RPW, 8)

        @pl.when(sid < NS - 1)
        def _():
            pltpu.sync_copy(zero_hbm.at[pl.ds(roff, RPW)],
                            acc.at[pl.ds(roff, RPW)])

        @pl.when(sid == NS - 1)
        def _():
            loff = pl.multiple_of((NS - 1) * RPW, 8)
            pltpu.sync_copy(zero_hbm.at[pl.ds(loff, RPW_LAST)],
                            acc.at[pl.ds(loff, RPW_LAST)])

        plsc.subcore_barrier()
        base = sid * EPS
        sis = (si0, si1, si2, si3)
        dis = (di0, di1, di2, di3)
        semis = (semi0, semi1, semi2, semi3)

        def issue_idx(j, k):
            off = base + j * C
            pltpu.async_copy(src_hbm.at[pl.ds(off, C)], sis[k], semis[k])
            pltpu.async_copy(dst_hbm.at[pl.ds(off, C)], dis[k], semis[k])

        def wait_idx(k):
            pltpu.make_async_copy(src_hbm.at[pl.ds(0, C)], sis[k],
                                  semis[k]).wait()
            pltpu.make_async_copy(dst_hbm.at[pl.ds(0, C)], dis[k],
                                  semis[k]).wait()

        def run(table, out):
            def start_gather(k, buf, semg):
                pltpu.async_copy(table.at[sis[k]], buf, semg)

            def wait_gather(buf, semg):
                pltpu.make_async_copy(zero_hbm.at[pl.ds(0, C)], buf,
                                      semg).wait()

            rbufs = (rows0, rows1)
            semgs = (semg0, semg1)
            for k in range(4):
                issue_idx(k, k)
            for k in range(2):
                wait_idx(k)
                start_gather(k, rbufs[k], semgs[k])

            def do_chunk(j, k, b):
                wait_gather(rbufs[b], semgs[b])
                pltpu.sync_copy(rbufs[b], acc.at[dis[k]], add=True)

                @pl.when(j + 4 < NCH)
                def _():
                    issue_idx(j + 4, k)

                @pl.when(j + 2 < NCH)
                def _():
                    wait_idx((k + 2) % 4)
                    start_gather((k + 2) % 4, rbufs[b], semgs[b])

            @pl.loop(0, NCH // 4)
            def _(q):
                j = q * 4
                do_chunk(j, 0, 0)
                do_chunk(j + 1, 1, 1)
                do_chunk(j + 2, 2, 0)
                do_chunk(j + 3, 3, 1)

            plsc.subcore_barrier()

            @pl.when(sid < NS - 1)
            def _():
                pltpu.sync_copy(acc.at[pl.ds(roff, RPW)],
                                out.at[pl.ds(roff, RPW)])

            @pl.when(sid == NS - 1)
            def _():
                loff = pl.multiple_of((NS - 1) * RPW, 8)
                pltpu.sync_copy(acc.at[pl.ds(loff, RPW_LAST)],
                                out.at[pl.ds(loff, RPW_LAST)])

        @pl.when(cid == 0)
        def _():
            run(xs_a, agg_a)

        @pl.when(cid == 1)
        def _():
            run(xs_b, agg_b)

    return agg_kernel


def _dinv_call(degp, N):
    def body(degp_ref, dinv_ref):
        deg = jnp.sum(degp_ref[...], axis=0, keepdims=True) + 1.0
        dinv_ref[...] = lax.rsqrt(deg)

    return pl.pallas_call(
        body, out_shape=jax.ShapeDtypeStruct((1, N), jnp.float32))(degp)


def _prologue_call(x, dinv_col, N, F):
    def body(x_ref, dv_ref, a_ref, b_ref):
        xs = x_ref[...] * dv_ref[...]
        a_ref[...] = xs[:, :F]
        b_ref[...] = xs[:, F:]

    return pl.pallas_call(
        body,
        out_shape=[jax.ShapeDtypeStruct((N, F), jnp.float32),
                   jax.ShapeDtypeStruct((N, F), jnp.float32)])(x, dinv_col)


def _epilogue_call(x, agg_a, agg_b, dinv_col, batch3, W1, b1, W2, b2,
                   N, R, G, HID):
    nblk = N // R

    def body(x_ref, aa_ref, ab_ref, dv_ref, b_ref, W1_ref, b1_ref, W2_ref,
             b2_ref, node_ref, graph_ref, sums_ref, cnts_ref):
        i = pl.program_id(0)
        dv = dv_ref[...]
        agg = jnp.concatenate([aa_ref[...], ab_ref[...]], axis=1)
        Z = dv * (agg + dv * x_ref[...])
        H = jnp.tanh(
            jnp.dot(Z, W1_ref[...], preferred_element_type=jnp.float32)
            + b1_ref[...])
        node_ref[...] = H
        bat = b_ref[0]
        gid = lax.broadcasted_iota(jnp.int32, (G, R), 0)
        onehot = (bat == gid).astype(jnp.float32)
        psum = jnp.dot(onehot, H, preferred_element_type=jnp.float32)
        pcnt = jnp.sum(onehot, axis=1, keepdims=True)

        @pl.when(i == 0)
        def _():
            sums_ref[...] = psum
            cnts_ref[...] = jnp.broadcast_to(pcnt, (G, 128))

        @pl.when(i > 0)
        def _():
            sums_ref[...] += psum
            cnts_ref[...] += jnp.broadcast_to(pcnt, (G, 128))

        @pl.when(i == nblk - 1)
        def _():
            cnt = jnp.maximum(cnts_ref[:, :1], 1.0)
            mean = sums_ref[...] / cnt
            graph_ref[...] = jnp.tanh(
                jnp.dot(mean, W2_ref[...], preferred_element_type=jnp.float32)
                + b2_ref[...])

    F = agg_a.shape[1]
    IN = x.shape[1]
    return pl.pallas_call(
        body,
        grid=(nblk,),
        in_specs=[
            pl.BlockSpec((R, IN), lambda i: (i, 0)),
            pl.BlockSpec((R, F), lambda i: (i, 0)),
            pl.BlockSpec((R, F), lambda i: (i, 0)),
            pl.BlockSpec((R, 1), lambda i: (i, 0)),
            pl.BlockSpec((1, 1, R), lambda i: (i, 0, 0)),
            pl.BlockSpec((IN, HID), lambda i: (0, 0)),
            pl.BlockSpec((1, HID), lambda i: (0, 0)),
            pl.BlockSpec((HID, HID), lambda i: (0, 0)),
            pl.BlockSpec((1, HID), lambda i: (0, 0)),
        ],
        out_specs=[
            pl.BlockSpec((R, HID), lambda i: (i, 0)),
            pl.BlockSpec((G, HID), lambda i: (0, 0)),
        ],
        out_shape=[jax.ShapeDtypeStruct((N, HID), jnp.float32),
                   jax.ShapeDtypeStruct((G, HID), jnp.float32)],
        scratch_shapes=[pltpu.VMEM((G, HID), jnp.float32),
                        pltpu.VMEM((G, 128), jnp.float32)],
    )(x, agg_a, agg_b, dinv_col, batch3, W1, b1, W2, b2)


def _impl(x, edge_index, batch, W1, b1, W2, b2):
    N, IN = x.shape
    E = edge_index.shape[1]
    HID = W1.shape[1]
    G = 64
    F = IN // 2
    R = 1000

    ei = edge_index.astype(jnp.int32)
    src, dst = ei[0], ei[1]

    degp = _make_deg_kernel(E, N)(dst)
    dinv_col = _dinv_call(degp, N).reshape(N, 1)
    xs_a, xs_b = _prologue_call(x, dinv_col, N, F)
    zeros = jnp.zeros((N, F), jnp.float32)
    EP = ((E + 128 * NS * 4 - 1) // (128 * NS * 4)) * 128 * NS * 4
    src2 = jnp.concatenate([src, jnp.zeros((EP - E,), jnp.int32)])
    dst2 = jnp.concatenate([dst, jnp.full((EP - E,), N, jnp.int32)])
    agg_a, agg_b = _make_agg_kernel(EP, N, F)(xs_a, xs_b, src2, dst2, zeros)

    batch3 = batch.astype(jnp.int32).reshape(N // R, 1, R)
    node, graph = _epilogue_call(
        x, agg_a, agg_b, dinv_col, batch3,
        W1, b1.reshape(1, HID), W2, b2.reshape(1, HID), N, R, G, HID)
    return (graph, node)


kernel = jax.jit(_impl)
